# baseline (speedup 1.0000x reference)
import functools

import jax
import jax.numpy as jnp
from jax import lax
from jax.experimental import pallas as pl
from jax.experimental.pallas import tpu as pltpu
from jax.experimental.pallas import tpu_sc as plsc

B = 16384
D = 64
WIDTH = 128
NE = 1000000
NR = 1000

_info = plsc.get_sparse_core_info()
_NC, _NS = _info.num_cores, _info.num_subcores
_NW = _NC * _NS
_BPW = B // _NW
_K = 16
_NCH = _BPW // _K


def _gather_rows(idx_hbm, tab_hbm, out_hbm, base, idx_v, rows_v, sem):
    _HB = _BPW // 2
    _HCH = _HB // _K
    pltpu.sync_copy(idx_hbm.at[pl.ds(base, _BPW)], idx_v)

    for h in range(2):
        hoff = h * _HB

        def fire(ci, hoff=hoff):
            ivec = idx_v[pl.ds(hoff + ci * _K, _K)]
            for j in range(_K):
                s = ivec[j]
                pltpu.async_copy(
                    tab_hbm.at[jnp.right_shift(s, 3),
                               pl.ds(jnp.bitwise_and(s, 7), 1), :],
                    rows_v.at[pl.ds(ci * _K + j, 1), :],
                    sem,
                )

        def drain(ci):
            for j in range(_K):
                pltpu.make_async_copy(
                    tab_hbm.at[0, pl.ds(0, 1), :],
                    rows_v.at[pl.ds(ci * _K + j, 1), :],
                    sem,
                ).wait()

        fire(0)

        def body(ci, c):
            fire(ci)
            drain(ci - 1)
            return c

        lax.fori_loop(1, _HCH, body, 0)
        drain(_HCH - 1)
        pltpu.sync_copy(rows_v, out_hbm.at[pl.ds(base + hoff, _HB)])


def _sc_gather_body(e1_hbm, rel_hbm, e2_hbm, ent_hbm, reltab_hbm,
                    head_hbm, relr_hbm, tail_hbm,
                    idx_v, rows_v, pairs_v, sem):
    wid = lax.axis_index("s") * _NC + lax.axis_index("c")
    base = wid * _BPW
    _HB = _BPW // 2
    _HCH = _HB // _K

    pltpu.sync_copy(e1_hbm.at[pl.ds(base, _BPW)], idx_v.at[0])
    pltpu.sync_copy(e2_hbm.at[pl.ds(base, _BPW)], idx_v.at[1])

    for h in range(2):
        hoff = h * _HB

        def fire(t, ci, hoff=hoff):
            ivec = idx_v[t, pl.ds(hoff + ci * _K, _K)]
            for j in range(_K):
                s = ivec[j]
                pltpu.async_copy(
                    ent_hbm.at[jnp.right_shift(s, 3),
                               pl.ds(jnp.bitwise_and(s, 7), 1), :],
                    rows_v.at[t, pl.ds(ci * _K + j, 1), :],
                    sem,
                )

        def drain(t, ci):
            for j in range(_K):
                pltpu.make_async_copy(
                    ent_hbm.at[0, pl.ds(0, 1), :],
                    rows_v.at[t, pl.ds(ci * _K + j, 1), :],
                    sem,
                ).wait()

        fire(0, 0)
        fire(1, 0)

        def body(ci, c):
            fire(0, ci)
            drain(0, ci - 1)
            fire(1, ci)
            drain(1, ci - 1)
            return c

        lax.fori_loop(1, _HCH, body, 0)
        drain(0, _HCH - 1)
        drain(1, _HCH - 1)
        pltpu.sync_copy(rows_v.at[0], head_hbm.at[pl.ds(base + hoff, _HB)])
        pltpu.sync_copy(rows_v.at[1], tail_hbm.at[pl.ds(base + hoff, _HB)])

    pltpu.sync_copy(rel_hbm.at[pl.ds(base, _BPW)], idx_v.at[0])

    for h in range(2):
        hoff = h * _HB

        def rfire(ci, hoff=hoff):
            ivec = idx_v[0, pl.ds(hoff + ci * _K, _K)]
            qvec = jnp.right_shift(ivec, 1)
            pltpu.async_copy(
                reltab_hbm.at[qvec],
                pairs_v.at[ci % 2],
                sem,
            )

        def rdrain(ci):
            pltpu.make_async_copy(
                reltab_hbm.at[pl.ds(0, _K)],
                pairs_v.at[ci % 2],
                sem,
            ).wait()

        def rextract(ci, hoff=hoff):
            slot = ci % 2
            ivec = idx_v[0, pl.ds(hoff + ci * _K, _K)]
            pvec = jnp.bitwise_and(ivec, 1) * D
            for j in range(_K):
                off = pvec[j]
                r = ci * _K + j
                for k in range(D // 16):
                    rows_v[0, r, pl.ds(k * 16, 16)] = (
                        pairs_v[slot, j, pl.ds(off + k * 16, 16)])

        rfire(0)

        def rbody(ci, c):
            rfire(ci)
            rdrain(ci - 1)
            rextract(ci - 1)
            return c

        lax.fori_loop(1, _HCH, rbody, 0)
        rdrain(_HCH - 1)
        rextract(_HCH - 1)
        pltpu.sync_copy(rows_v.at[0], relr_hbm.at[pl.ds(base + hoff, _HB)])


_sc_gather = functools.partial(
    pl.kernel,
    mesh=plsc.VectorSubcoreMesh(core_axis_name="c", subcore_axis_name="s"),
    out_type=[jax.ShapeDtypeStruct((B, D), jnp.float32)] * 3,
    scratch_types=[
        pltpu.VMEM((2, _BPW), jnp.int32),
        pltpu.VMEM((2, _BPW // 2, D), jnp.float32),
        pltpu.VMEM((2, _K, 2 * D), jnp.float32),
        pltpu.SemaphoreType.DMA,
    ],
)(_sc_gather_body)


_BB = 2048
_NB = B // _BB


def _mlp_body(head_ref, rel_ref, tail_ref, w1_ref, b1_ref, gamma_ref,
              beta_ref, w2_ref, b2_ref, out_ref, x_buf, stat_ref):
    p = pl.program_id(0)
    i = pl.program_id(1)

    @pl.when(p == 0)
    def _pass0():
        w1 = w1_ref[...]
        x = jnp.dot(head_ref[...], w1[0:D, :],
                    preferred_element_type=jnp.float32)
        x += jnp.dot(rel_ref[...], w1[D:2 * D, :],
                     preferred_element_type=jnp.float32)
        x += jnp.dot(tail_ref[...], w1[2 * D:3 * D, :],
                     preferred_element_type=jnp.float32)
        x += b1_ref[...]
        x_buf[pl.ds(i * _BB, _BB), :] = x
        s = jnp.sum(x, axis=0, keepdims=True)
        sq = jnp.sum(x * x, axis=0, keepdims=True)

        @pl.when(i == 0)
        def _():
            stat_ref[0:1, :] = s
            stat_ref[1:2, :] = sq

        @pl.when(i > 0)
        def _():
            stat_ref[0:1, :] = stat_ref[0:1, :] + s
            stat_ref[1:2, :] = stat_ref[1:2, :] + sq

    @pl.when(p == 1)
    def _pass1():
        mean = stat_ref[0:1, :] * (1.0 / B)
        var = stat_ref[1:2, :] * (1.0 / B) - mean * mean
        inv = lax.rsqrt(var + 1e-5)
        x = x_buf[pl.ds(i * _BB, _BB), :]
        xn = (x - mean) * (inv * gamma_ref[...]) + beta_ref[...]
        xn = jnp.maximum(xn, 0.0)
        y = jnp.dot(xn, w2_ref[...], preferred_element_type=jnp.float32)
        y += b2_ref[...]
        out_ref[...] = jax.nn.sigmoid(y)


def _mlp(head, relr, tail, W1, b1, gamma, beta, W2, b2):
    grid = (2, _NB)
    return pl.pallas_call(
        _mlp_body,
        grid=grid,
        in_specs=[
            pl.BlockSpec((_BB, D), lambda p, i: (i * (1 - p), 0)),
            pl.BlockSpec((_BB, D), lambda p, i: (i * (1 - p), 0)),
            pl.BlockSpec((_BB, D), lambda p, i: (i * (1 - p), 0)),
            pl.BlockSpec((3 * D, WIDTH), lambda p, i: (0, 0)),
            pl.BlockSpec((1, WIDTH), lambda p, i: (0, 0)),
            pl.BlockSpec((1, WIDTH), lambda p, i: (0, 0)),
            pl.BlockSpec((1, WIDTH), lambda p, i: (0, 0)),
            pl.BlockSpec((WIDTH, 1), lambda p, i: (0, 0)),
            pl.BlockSpec((1, 1), lambda p, i: (0, 0)),
        ],
        out_specs=pl.BlockSpec((_BB, 1), lambda p, i: (i, 0)),
        out_shape=jax.ShapeDtypeStruct((B, 1), jnp.float32),
        scratch_shapes=[
            pltpu.VMEM((B, WIDTH), jnp.float32),
            pltpu.VMEM((8, WIDTH), jnp.float32),
        ],
    )(head, relr, tail, W1, b1, gamma, beta, W2, b2)


def kernel(e1_idx, rel_idx, e2_idx, entity_emb, relation_emb, W1, b1, gamma,
           beta, W2, b2):
    e1 = e1_idx.astype(jnp.int32)
    rel = rel_idx.astype(jnp.int32)
    e2 = e2_idx.astype(jnp.int32)
    ent3 = entity_emb.reshape(NE // 8, 8, D)
    rel3 = relation_emb.reshape(NR // 2, 2 * D)
    head, relr, tail = _sc_gather(e1, rel, e2, ent3, rel3)
    return _mlp(head, relr, tail, W1,
                b1.reshape(1, WIDTH), gamma.reshape(1, WIDTH),
                beta.reshape(1, WIDTH), W2, b2.reshape(1, 1))


# speedup vs baseline: 1.0084x; 1.0084x over previous
import functools

import jax
import jax.numpy as jnp
from jax import lax
from jax.experimental import pallas as pl
from jax.experimental.pallas import tpu as pltpu
from jax.experimental.pallas import tpu_sc as plsc

B = 16384
D = 64
WIDTH = 128
NE = 1000000
NR = 1000

_info = plsc.get_sparse_core_info()
_NC, _NS = _info.num_cores, _info.num_subcores
_NW = _NC * _NS
_BPW = B // _NW
_K = 16
_NCH = _BPW // _K


def _gather_rows(idx_hbm, tab_hbm, out_hbm, base, idx_v, rows_v, sem):
    _HB = _BPW // 2
    _HCH = _HB // _K
    pltpu.sync_copy(idx_hbm.at[pl.ds(base, _BPW)], idx_v)

    for h in range(2):
        hoff = h * _HB

        def fire(ci, hoff=hoff):
            ivec = idx_v[pl.ds(hoff + ci * _K, _K)]
            for j in range(_K):
                s = ivec[j]
                pltpu.async_copy(
                    tab_hbm.at[jnp.right_shift(s, 3),
                               pl.ds(jnp.bitwise_and(s, 7), 1), :],
                    rows_v.at[pl.ds(ci * _K + j, 1), :],
                    sem,
                )

        def drain(ci):
            for j in range(_K):
                pltpu.make_async_copy(
                    tab_hbm.at[0, pl.ds(0, 1), :],
                    rows_v.at[pl.ds(ci * _K + j, 1), :],
                    sem,
                ).wait()

        fire(0)

        def body(ci, c):
            fire(ci)
            drain(ci - 1)
            return c

        lax.fori_loop(1, _HCH, body, 0)
        drain(_HCH - 1)
        pltpu.sync_copy(rows_v, out_hbm.at[pl.ds(base + hoff, _HB)])


def _sc_gather_body(e1_hbm, rel_hbm, e2_hbm, ent_hbm, reltab_hbm,
                    head_hbm, relr_hbm, tail_hbm,
                    idx_v, rows_v, sem):
    wid = lax.axis_index("s") * _NC + lax.axis_index("c")
    base = wid * _BPW
    _HB = _BPW // 2
    _HCH = _HB // _K

    pltpu.sync_copy(e1_hbm.at[pl.ds(base, _BPW)], idx_v.at[0])
    pltpu.sync_copy(e2_hbm.at[pl.ds(base, _BPW)], idx_v.at[1])

    for h in range(2):
        hoff = h * _HB

        def fire(t, ci, hoff=hoff):
            ivec = idx_v[t, pl.ds(hoff + ci * _K, _K)]
            for j in range(_K):
                s = ivec[j]
                pltpu.async_copy(
                    ent_hbm.at[jnp.right_shift(s, 3),
                               pl.ds(jnp.bitwise_and(s, 7), 1), :],
                    rows_v.at[t, pl.ds(ci * _K + j, 1), :],
                    sem,
                )

        def drain(t, ci):
            for j in range(_K):
                pltpu.make_async_copy(
                    ent_hbm.at[0, pl.ds(0, 1), :],
                    rows_v.at[t, pl.ds(ci * _K + j, 1), :],
                    sem,
                ).wait()

        fire(0, 0)
        fire(1, 0)

        def body(ci, c):
            fire(0, ci)
            drain(0, ci - 1)
            fire(1, ci)
            drain(1, ci - 1)
            return c

        lax.fori_loop(1, _HCH, body, 0)
        drain(0, _HCH - 1)
        drain(1, _HCH - 1)
        pltpu.sync_copy(rows_v.at[0], head_hbm.at[pl.ds(base + hoff, _HB)])
        pltpu.sync_copy(rows_v.at[1], tail_hbm.at[pl.ds(base + hoff, _HB)])

    _gather_rows(rel_hbm, reltab_hbm, relr_hbm, base, idx_v.at[0],
                 rows_v.at[0], sem)


_sc_gather = functools.partial(
    pl.kernel,
    mesh=plsc.VectorSubcoreMesh(core_axis_name="c", subcore_axis_name="s"),
    out_type=[jax.ShapeDtypeStruct((B, D), jnp.float32)] * 3,
    scratch_types=[
        pltpu.VMEM((2, _BPW), jnp.int32),
        pltpu.VMEM((2, _BPW // 2, D), jnp.float32),
        pltpu.SemaphoreType.DMA,
    ],
)(_sc_gather_body)


_BB = 2048
_NB = B // _BB


def _mlp_body(head_ref, rel_ref, tail_ref, w1_ref, b1_ref, gamma_ref,
              beta_ref, w2_ref, b2_ref, out_ref, x_buf, stat_ref):
    p = pl.program_id(0)
    i = pl.program_id(1)

    @pl.when(p == 0)
    def _pass0():
        w1 = w1_ref[...]
        x = jnp.dot(head_ref[...], w1[0:D, :],
                    preferred_element_type=jnp.float32)
        x += jnp.dot(rel_ref[...], w1[D:2 * D, :],
                     preferred_element_type=jnp.float32)
        x += jnp.dot(tail_ref[...], w1[2 * D:3 * D, :],
                     preferred_element_type=jnp.float32)
        x += b1_ref[...]
        x_buf[pl.ds(i * _BB, _BB), :] = x
        s = jnp.sum(x, axis=0, keepdims=True)
        sq = jnp.sum(x * x, axis=0, keepdims=True)

        @pl.when(i == 0)
        def _():
            stat_ref[0:1, :] = s
            stat_ref[1:2, :] = sq

        @pl.when(i > 0)
        def _():
            stat_ref[0:1, :] = stat_ref[0:1, :] + s
            stat_ref[1:2, :] = stat_ref[1:2, :] + sq

    @pl.when(p == 1)
    def _pass1():
        mean = stat_ref[0:1, :] * (1.0 / B)
        var = stat_ref[1:2, :] * (1.0 / B) - mean * mean
        inv = lax.rsqrt(var + 1e-5)
        x = x_buf[pl.ds(i * _BB, _BB), :]
        xn = (x - mean) * (inv * gamma_ref[...]) + beta_ref[...]
        xn = jnp.maximum(xn, 0.0)
        y = jnp.dot(xn, w2_ref[...], preferred_element_type=jnp.float32)
        y += b2_ref[...]
        out_ref[...] = jax.nn.sigmoid(y)


def _mlp(head, relr, tail, W1, b1, gamma, beta, W2, b2):
    grid = (2, _NB)
    return pl.pallas_call(
        _mlp_body,
        grid=grid,
        in_specs=[
            pl.BlockSpec((_BB, D), lambda p, i: (i * (1 - p), 0)),
            pl.BlockSpec((_BB, D), lambda p, i: (i * (1 - p), 0)),
            pl.BlockSpec((_BB, D), lambda p, i: (i * (1 - p), 0)),
            pl.BlockSpec((3 * D, WIDTH), lambda p, i: (0, 0)),
            pl.BlockSpec((1, WIDTH), lambda p, i: (0, 0)),
            pl.BlockSpec((1, WIDTH), lambda p, i: (0, 0)),
            pl.BlockSpec((1, WIDTH), lambda p, i: (0, 0)),
            pl.BlockSpec((WIDTH, 1), lambda p, i: (0, 0)),
            pl.BlockSpec((1, 1), lambda p, i: (0, 0)),
        ],
        out_specs=pl.BlockSpec((_BB, 1), lambda p, i: (i, 0)),
        out_shape=jax.ShapeDtypeStruct((B, 1), jnp.float32),
        scratch_shapes=[
            pltpu.VMEM((B, WIDTH), jnp.float32),
            pltpu.VMEM((8, WIDTH), jnp.float32),
        ],
    )(head, relr, tail, W1, b1, gamma, beta, W2, b2)


def kernel(e1_idx, rel_idx, e2_idx, entity_emb, relation_emb, W1, b1, gamma,
           beta, W2, b2):
    e1 = e1_idx.astype(jnp.int32)
    rel = rel_idx.astype(jnp.int32)
    e2 = e2_idx.astype(jnp.int32)
    ent3 = entity_emb.reshape(NE // 8, 8, D)
    rel3 = relation_emb.reshape(NR // 8, 8, D)
    head, relr, tail = _sc_gather(e1, rel, e2, ent3, rel3)
    return _mlp(head, relr, tail, W1,
                b1.reshape(1, WIDTH), gamma.reshape(1, WIDTH),
                beta.reshape(1, WIDTH), W2, b2.reshape(1, 1))


# speedup vs baseline: 1.0141x; 1.0056x over previous
import functools

import jax
import jax.numpy as jnp
from jax import lax
from jax.experimental import pallas as pl
from jax.experimental.pallas import tpu as pltpu
from jax.experimental.pallas import tpu_sc as plsc

B = 16384
D = 64
WIDTH = 128
NE = 1000000
NR = 1000

_info = plsc.get_sparse_core_info()
_NC, _NS = _info.num_cores, _info.num_subcores
_NW = _NC * _NS
_BPW = B // _NW
_K = 16
_NCH = _BPW // _K


def _gather_rows(idx_hbm, tab_hbm, out_hbm, base, idx_v, rows_v, sem):
    _HB = _BPW // 2
    _HCH = _HB // _K
    pltpu.sync_copy(idx_hbm.at[pl.ds(base, _BPW)], idx_v)

    for h in range(2):
        hoff = h * _HB

        def fire(ci, hoff=hoff):
            ivec = idx_v[pl.ds(hoff + ci * _K, _K)]
            for j in range(_K):
                s = ivec[j]
                pltpu.async_copy(
                    tab_hbm.at[jnp.right_shift(s, 3),
                               pl.ds(jnp.bitwise_and(s, 7), 1), :],
                    rows_v.at[pl.ds(ci * _K + j, 1), :],
                    sem,
                )

        def drain(ci):
            for j in range(_K):
                pltpu.make_async_copy(
                    tab_hbm.at[0, pl.ds(0, 1), :],
                    rows_v.at[pl.ds(ci * _K + j, 1), :],
                    sem,
                ).wait()

        fire(0)

        def body(ci, c):
            fire(ci)
            drain(ci - 1)
            return c

        lax.fori_loop(1, _HCH, body, 0)
        drain(_HCH - 1)
        pltpu.sync_copy(rows_v, out_hbm.at[pl.ds(base + hoff, _HB)])


def _sc_gather_body(e1_hbm, rel_hbm, e2_hbm, ent_hbm, reltab_hbm,
                    head_hbm, relr_hbm, tail_hbm,
                    idx_v, rows_v, sem):
    wid = lax.axis_index("s") * _NC + lax.axis_index("c")
    base = wid * _BPW
    _HB = _BPW // 2
    _HCH = _HB // _K

    pltpu.sync_copy(e1_hbm.at[pl.ds(base, _BPW)], idx_v.at[0])
    pltpu.sync_copy(e2_hbm.at[pl.ds(base, _BPW)], idx_v.at[1])

    for h in range(2):
        hoff = h * _HB

        def fire(t, ci, hoff=hoff):
            ivec = idx_v[t, pl.ds(hoff + ci * _K, _K)]
            for j in range(_K):
                s = ivec[j]
                pltpu.async_copy(
                    ent_hbm.at[jnp.right_shift(s, 3),
                               pl.ds(jnp.bitwise_and(s, 7), 1), :],
                    rows_v.at[t, pl.ds(ci * _K + j, 1), :],
                    sem,
                )

        def drain(t, ci):
            for j in range(_K):
                pltpu.make_async_copy(
                    ent_hbm.at[0, pl.ds(0, 1), :],
                    rows_v.at[t, pl.ds(ci * _K + j, 1), :],
                    sem,
                ).wait()

        fire(0, 0)
        fire(1, 0)

        def body(ci, c):
            fire(0, ci)
            drain(0, ci - 1)
            fire(1, ci)
            drain(1, ci - 1)
            return c

        lax.fori_loop(1, _HCH, body, 0)
        drain(0, _HCH - 1)
        drain(1, _HCH - 1)
        pltpu.sync_copy(rows_v.at[0], head_hbm.at[pl.ds(base + hoff, _HB)])
        pltpu.sync_copy(rows_v.at[1], tail_hbm.at[pl.ds(base + hoff, _HB)])

    _gather_rows(rel_hbm, reltab_hbm, relr_hbm, base, idx_v.at[0],
                 rows_v.at[0], sem)


_sc_gather = functools.partial(
    pl.kernel,
    mesh=plsc.VectorSubcoreMesh(core_axis_name="c", subcore_axis_name="s"),
    out_type=[jax.ShapeDtypeStruct((B, D), jnp.float32)] * 3,
    scratch_types=[
        pltpu.VMEM((2, _BPW), jnp.int32),
        pltpu.VMEM((2, _BPW // 2, D), jnp.float32),
        pltpu.SemaphoreType.DMA,
    ],
)(_sc_gather_body)


_BB = 4096
_NB = B // _BB


def _mlp_body(head_ref, rel_ref, tail_ref, w1_ref, b1_ref, gamma_ref,
              beta_ref, w2_ref, b2_ref, out_ref, x_buf, stat_ref):
    p = pl.program_id(0)
    i = pl.program_id(1)

    @pl.when(p == 0)
    def _pass0():
        w1 = w1_ref[...]
        x = jnp.dot(head_ref[...], w1[0:D, :],
                    preferred_element_type=jnp.float32)
        x += jnp.dot(rel_ref[...], w1[D:2 * D, :],
                     preferred_element_type=jnp.float32)
        x += jnp.dot(tail_ref[...], w1[2 * D:3 * D, :],
                     preferred_element_type=jnp.float32)
        x += b1_ref[...]
        x_buf[pl.ds(i * _BB, _BB), :] = x
        s = jnp.sum(x, axis=0, keepdims=True)
        sq = jnp.sum(x * x, axis=0, keepdims=True)

        @pl.when(i == 0)
        def _():
            stat_ref[0:1, :] = s
            stat_ref[1:2, :] = sq

        @pl.when(i > 0)
        def _():
            stat_ref[0:1, :] = stat_ref[0:1, :] + s
            stat_ref[1:2, :] = stat_ref[1:2, :] + sq

    @pl.when(p == 1)
    def _pass1():
        mean = stat_ref[0:1, :] * (1.0 / B)
        var = stat_ref[1:2, :] * (1.0 / B) - mean * mean
        inv = lax.rsqrt(var + 1e-5)
        x = x_buf[pl.ds(i * _BB, _BB), :]
        xn = (x - mean) * (inv * gamma_ref[...]) + beta_ref[...]
        xn = jnp.maximum(xn, 0.0)
        y = jnp.dot(xn, w2_ref[...], preferred_element_type=jnp.float32)
        y += b2_ref[...]
        out_ref[...] = jax.nn.sigmoid(y)


def _mlp(head, relr, tail, W1, b1, gamma, beta, W2, b2):
    grid = (2, _NB)
    return pl.pallas_call(
        _mlp_body,
        grid=grid,
        in_specs=[
            pl.BlockSpec((_BB, D), lambda p, i: (i * (1 - p), 0)),
            pl.BlockSpec((_BB, D), lambda p, i: (i * (1 - p), 0)),
            pl.BlockSpec((_BB, D), lambda p, i: (i * (1 - p), 0)),
            pl.BlockSpec((3 * D, WIDTH), lambda p, i: (0, 0)),
            pl.BlockSpec((1, WIDTH), lambda p, i: (0, 0)),
            pl.BlockSpec((1, WIDTH), lambda p, i: (0, 0)),
            pl.BlockSpec((1, WIDTH), lambda p, i: (0, 0)),
            pl.BlockSpec((WIDTH, 1), lambda p, i: (0, 0)),
            pl.BlockSpec((1, 1), lambda p, i: (0, 0)),
        ],
        out_specs=pl.BlockSpec((_BB, 1), lambda p, i: (i, 0)),
        out_shape=jax.ShapeDtypeStruct((B, 1), jnp.float32),
        scratch_shapes=[
            pltpu.VMEM((B, WIDTH), jnp.float32),
            pltpu.VMEM((8, WIDTH), jnp.float32),
        ],
    )(head, relr, tail, W1, b1, gamma, beta, W2, b2)


def kernel(e1_idx, rel_idx, e2_idx, entity_emb, relation_emb, W1, b1, gamma,
           beta, W2, b2):
    e1 = e1_idx.astype(jnp.int32)
    rel = rel_idx.astype(jnp.int32)
    e2 = e2_idx.astype(jnp.int32)
    ent3 = entity_emb.reshape(NE // 8, 8, D)
    rel3 = relation_emb.reshape(NR // 8, 8, D)
    head, relr, tail = _sc_gather(e1, rel, e2, ent3, rel3)
    return _mlp(head, relr, tail, W1,
                b1.reshape(1, WIDTH), gamma.reshape(1, WIDTH),
                beta.reshape(1, WIDTH), W2, b2.reshape(1, 1))


# speedup vs baseline: 1.0286x; 1.0143x over previous
import functools

import jax
import jax.numpy as jnp
from jax import lax
from jax.experimental import pallas as pl
from jax.experimental.pallas import tpu as pltpu
from jax.experimental.pallas import tpu_sc as plsc

B = 16384
D = 64
WIDTH = 128
NE = 1000000
NR = 1000

_info = plsc.get_sparse_core_info()
_NC, _NS = _info.num_cores, _info.num_subcores
_NW = _NC * _NS
_BPW = B // _NW
_K = 16
_NCH = _BPW // _K


def _gather_rows(idx_hbm, tab_hbm, out_hbm, base, idx_v, rows_v, sem):
    _HB = _BPW // 2
    _HCH = _HB // _K
    pltpu.sync_copy(idx_hbm.at[pl.ds(base, _BPW)], idx_v)

    for h in range(2):
        hoff = h * _HB

        def fire(ci, hoff=hoff):
            ivec = idx_v[pl.ds(hoff + ci * _K, _K)]
            for j in range(_K):
                s = ivec[j]
                pltpu.async_copy(
                    tab_hbm.at[jnp.right_shift(s, 3),
                               pl.ds(jnp.bitwise_and(s, 7), 1), :],
                    rows_v.at[pl.ds(ci * _K + j, 1), :],
                    sem,
                )

        def drain(ci):
            for j in range(_K):
                pltpu.make_async_copy(
                    tab_hbm.at[0, pl.ds(0, 1), :],
                    rows_v.at[pl.ds(ci * _K + j, 1), :],
                    sem,
                ).wait()

        fire(0)

        def body(ci, c):
            fire(ci)
            drain(ci - 1)
            return c

        lax.fori_loop(1, _HCH, body, 0)
        drain(_HCH - 1)
        pltpu.sync_copy(rows_v, out_hbm.at[pl.ds(base + hoff, _HB)])


def _sc_gather_body(e1_hbm, rel_hbm, e2_hbm, ent_hbm, reltab_hbm,
                    head_hbm, relr_hbm, tail_hbm,
                    idx_v, rows_v, sem):
    wid = lax.axis_index("s") * _NC + lax.axis_index("c")
    base = wid * _BPW
    _HB = _BPW // 2
    _HCH = _HB // _K

    pltpu.sync_copy(e1_hbm.at[pl.ds(base, _BPW)], idx_v.at[0])
    pltpu.sync_copy(e2_hbm.at[pl.ds(base, _BPW)], idx_v.at[1])

    for h in range(2):
        hoff = h * _HB

        def fire(t, ci, hoff=hoff):
            ivec = idx_v[t, pl.ds(hoff + ci * _K, _K)]
            for j in range(_K):
                s = ivec[j]
                pltpu.async_copy(
                    ent_hbm.at[jnp.right_shift(s, 3),
                               pl.ds(jnp.bitwise_and(s, 7), 1), :],
                    rows_v.at[t, pl.ds(ci * _K + j, 1), :],
                    sem,
                )

        def drain(t, ci):
            for j in range(_K):
                pltpu.make_async_copy(
                    ent_hbm.at[0, pl.ds(0, 1), :],
                    rows_v.at[t, pl.ds(ci * _K + j, 1), :],
                    sem,
                ).wait()

        fire(0, 0)
        fire(1, 0)

        def body(ci, c):
            fire(0, ci)
            drain(0, ci - 1)
            fire(1, ci)
            drain(1, ci - 1)
            return c

        lax.fori_loop(1, _HCH, body, 0)
        drain(0, _HCH - 1)
        drain(1, _HCH - 1)
        pltpu.sync_copy(rows_v.at[0], head_hbm.at[pl.ds(base + hoff, _HB)])
        pltpu.sync_copy(rows_v.at[1], tail_hbm.at[pl.ds(base + hoff, _HB)])

    _gather_rows(rel_hbm, reltab_hbm, relr_hbm, base, idx_v.at[0],
                 rows_v.at[0], sem)


_sc_gather = functools.partial(
    pl.kernel,
    mesh=plsc.VectorSubcoreMesh(core_axis_name="c", subcore_axis_name="s"),
    out_type=[jax.ShapeDtypeStruct((B, D), jnp.float32)] * 3,
    scratch_types=[
        pltpu.VMEM((2, _BPW), jnp.int32),
        pltpu.VMEM((2, _BPW // 2, D), jnp.float32),
        pltpu.SemaphoreType.DMA,
    ],
)(_sc_gather_body)


_BB = 4096
_NB = B // _BB


def _mlp_body(head_ref, rel_ref, tail_ref, w1_ref, b1_ref, gamma_ref,
              beta_ref, w2_ref, b2_ref, out_ref, x_buf, stat_ref):
    p = pl.program_id(0)
    i = pl.program_id(1)

    @pl.when(p == 0)
    def _pass0():
        w1 = w1_ref[...]
        x = jnp.dot(head_ref[...], w1[0:D, :],
                    preferred_element_type=jnp.float32)
        x += jnp.dot(rel_ref[...], w1[D:2 * D, :],
                     preferred_element_type=jnp.float32)
        x += jnp.dot(tail_ref[...], w1[2 * D:3 * D, :],
                     preferred_element_type=jnp.float32)
        x += b1_ref[...]
        x_buf[pl.ds(i * _BB, _BB), :] = x
        s = jnp.sum(x, axis=0, keepdims=True)
        sq = jnp.sum(x * x, axis=0, keepdims=True)

        @pl.when(i == 0)
        def _():
            stat_ref[0:1, :] = s
            stat_ref[1:2, :] = sq

        @pl.when(i > 0)
        def _():
            stat_ref[0:1, :] = stat_ref[0:1, :] + s
            stat_ref[1:2, :] = stat_ref[1:2, :] + sq

    @pl.when(p == 1)
    def _pass1():
        mean = stat_ref[0:1, :] * (1.0 / B)
        var = stat_ref[1:2, :] * (1.0 / B) - mean * mean
        inv = lax.rsqrt(var + 1e-5)
        x = x_buf[pl.ds(i * _BB, _BB), :]
        xn = (x - mean) * (inv * gamma_ref[...]) + beta_ref[...]
        xn = jnp.maximum(xn, 0.0)
        y = jnp.sum(xn * w2_ref[...], axis=1) + b2_ref[0, 0]
        out_ref[...] = jax.nn.sigmoid(y)


def _mlp(head, relr, tail, W1, b1, gamma, beta, W2, b2):
    grid = (2, _NB)
    return pl.pallas_call(
        _mlp_body,
        grid=grid,
        in_specs=[
            pl.BlockSpec((_BB, D), lambda p, i: (i * (1 - p), 0)),
            pl.BlockSpec((_BB, D), lambda p, i: (i * (1 - p), 0)),
            pl.BlockSpec((_BB, D), lambda p, i: (i * (1 - p), 0)),
            pl.BlockSpec((3 * D, WIDTH), lambda p, i: (0, 0)),
            pl.BlockSpec((1, WIDTH), lambda p, i: (0, 0)),
            pl.BlockSpec((1, WIDTH), lambda p, i: (0, 0)),
            pl.BlockSpec((1, WIDTH), lambda p, i: (0, 0)),
            pl.BlockSpec((1, WIDTH), lambda p, i: (0, 0)),
            pl.BlockSpec((1, 1), lambda p, i: (0, 0)),
        ],
        out_specs=pl.BlockSpec((_BB,), lambda p, i: (i,)),
        out_shape=jax.ShapeDtypeStruct((B,), jnp.float32),
        scratch_shapes=[
            pltpu.VMEM((B, WIDTH), jnp.float32),
            pltpu.VMEM((8, WIDTH), jnp.float32),
        ],
    )(head, relr, tail, W1, b1, gamma, beta, W2, b2)


def kernel(e1_idx, rel_idx, e2_idx, entity_emb, relation_emb, W1, b1, gamma,
           beta, W2, b2):
    e1 = e1_idx.astype(jnp.int32)
    rel = rel_idx.astype(jnp.int32)
    e2 = e2_idx.astype(jnp.int32)
    ent3 = entity_emb.reshape(NE // 8, 8, D)
    rel3 = relation_emb.reshape(NR // 8, 8, D)
    head, relr, tail = _sc_gather(e1, rel, e2, ent3, rel3)
    out = _mlp(head, relr, tail, W1,
               b1.reshape(1, WIDTH), gamma.reshape(1, WIDTH),
               beta.reshape(1, WIDTH), W2.reshape(1, WIDTH),
               b2.reshape(1, 1))
    return out.reshape(B, 1)
